# baseline (speedup 1.0000x reference)
import functools

import jax
import jax.numpy as jnp
from jax import lax
from jax.experimental import pallas as pl
from jax.experimental.pallas import tpu as pltpu
from jax.experimental.pallas import tpu_sc as plsc

B = 16384
F = 32
FT = 4
L = 16
TW = 128
NC = 2
NS = 16
NW = NC * NS
BPW = B // NW
K = 16
NCHUNK = BPW // K
NTB = 782
TROWS = NTB * TW
TTR = TROWS // 8


def _sc_body(time_hbm, item_hbm, tf_hbm, if_hbm, out_hbm, ttab_hbm,
             tidx_v, iidx_v, tgidx_v, blk_v, trowbuf_v, trows_v, out_v,
             sem, sem_s, sem_t):
    sid = lax.axis_index("s")
    wid = sid * NC + lax.axis_index("c")
    base = wid * BPW
    lane = lax.iota(jnp.int32, L)

    pltpu.sync_copy(time_hbm.at[pl.ds(base, BPW)], tidx_v)
    pltpu.sync_copy(item_hbm.at[pl.ds(base, BPW)], iidx_v)

    nj = (NTB - 1) // NS + 1

    def fire_stage(j):
        bid = sid + j * NS
        slot = j & 1

        @pl.when(bid < NTB)
        def _():
            for ft in range(FT):
                pltpu.async_copy(
                    tf_hbm.at[pl.ds(ft * 8, 8), pl.ds(bid * TW, TW)],
                    blk_v.at[slot * FT + ft], sem_s)

    def stage_block(j, carry):
        fire_stage(j + 1)
        bid = sid + j * NS
        slot = j & 1

        @pl.when(bid < NTB)
        def _():
            for ft in range(FT):
                pltpu.make_async_copy(
                    tf_hbm.at[pl.ds(0, 8), pl.ds(0, TW)],
                    blk_v.at[slot * FT + ft], sem_s,
                ).wait()
            rlo = slot * FT + (lane >> 3)
            rhi = rlo + 2
            fsub = lane & 7
            for lph in range(2):
                def lanebody(li, carry2):
                    l = lph * 64 + li
                    lv = jnp.full((L,), l, jnp.int32)
                    v1 = plsc.load_gather(blk_v, [rlo, fsub, lv])
                    v2 = plsc.load_gather(blk_v, [rhi, fsub, lv])
                    packed = plsc.pack(
                        v1, v2, format=plsc.PackFormat.INTERLEAVED)
                    trowbuf_v[li >> 3, pl.ds((li & 7) * L, L)] = (
                        plsc.bitcast(packed, jnp.int32))
                    return carry2

                lax.fori_loop(0, 64, lanebody, 0)
                pltpu.sync_copy(
                    trowbuf_v,
                    ttab_hbm.at[pl.ds(bid * (TW // 8) + lph * 8, 8)])
        return carry

    fire_stage(0)
    lax.fori_loop(0, nj, stage_block, 0)

    plsc.subcore_barrier()

    def tg(g, carry):
        s = pl.ds(g * L, L)
        tgidx_v[s] = tidx_v[s] >> 3
        return carry

    lax.fori_loop(0, BPW // L, tg, 0)
    pltpu.async_copy(ttab_hbm.at[tgidx_v], trows_v, sem_t).wait()

    def chunk(c, carry):
        idx_vec = iidx_v[pl.ds(c * K, L)]
        cols = (idx_vec >> 7) << 7
        lanes = idx_vec & (TW - 1)
        rows = c * K + lane
        tsub = (tidx_v[pl.ds(c * K, L)] & 7) << 4
        acc = jnp.zeros((L,), jnp.float32)
        for ft in range(FT):
            for r in range(K):
                col = pl.multiple_of(cols[r], TW)
                pltpu.async_copy(
                    if_hbm.at[pl.ds(ft * 8, 8), pl.ds(col, TW)],
                    blk_v.at[r], sem)

            def drain(r, carry2):
                pltpu.make_async_copy(
                    if_hbm.at[pl.ds(0, 8), pl.ds(0, TW)], blk_v.at[r], sem
                ).wait()
                return carry2

            lax.fori_loop(0, K, drain, 0, unroll=2)

            for p8 in range(8):
                f = ft * 8 + p8
                pv = jnp.full((L,), p8, jnp.int32)
                iv = plsc.load_gather(blk_v, [lane, pv, lanes])
                tp = plsc.load_gather(trows_v, [rows, tsub + (f % L)])
                t_lo, t_hi = plsc.unpack(
                    plsc.bitcast(tp, jnp.bfloat16),
                    format=plsc.PackFormat.INTERLEAVED)
                acc = acc + (t_lo if f < L else t_hi) * iv
        out_v[pl.ds(c * K, L)] = acc
        return carry

    lax.fori_loop(0, NCHUNK, chunk, 0)

    pltpu.sync_copy(out_v, out_hbm.at[pl.ds(base, BPW)])


@jax.jit
def _run(time, item, tfT, ifT):
    kern = pl.kernel(
        _sc_body,
        out_type=(
            jax.ShapeDtypeStruct((B,), jnp.float32),
            jax.ShapeDtypeStruct((TTR, 8 * L), jnp.int32),
        ),
        mesh=plsc.VectorSubcoreMesh(core_axis_name="c", subcore_axis_name="s"),
        compiler_params=pltpu.CompilerParams(needs_layout_passes=False),
        scratch_types=[
            pltpu.VMEM((BPW,), jnp.int32),
            pltpu.VMEM((BPW,), jnp.int32),
            pltpu.VMEM((BPW,), jnp.int32),
            pltpu.VMEM((K, 8, TW), jnp.float32),
            pltpu.VMEM((8, 8 * L), jnp.int32),
            pltpu.VMEM((BPW, 8 * L), jnp.int32),
            pltpu.VMEM((BPW,), jnp.float32),
            pltpu.SemaphoreType.DMA,
            pltpu.SemaphoreType.DMA,
            pltpu.SemaphoreType.DMA,
        ],
    )
    out, _ = kern(time, item, tfT, ifT)
    return out


def kernel(time, item, time_factor, item_factor, lag_factor):
    del lag_factor
    return _run(time.astype(jnp.int32), item.astype(jnp.int32),
                time_factor.T, item_factor.T)


# speedup vs baseline: 1.2344x; 1.2344x over previous
import functools

import jax
import jax.numpy as jnp
from jax import lax
from jax.experimental import pallas as pl
from jax.experimental.pallas import tpu as pltpu
from jax.experimental.pallas import tpu_sc as plsc

B = 16384
F = 32
FT = 4
L = 16
TW = 128
NC = 2
NS = 16
NW = NC * NS
BPW = B // NW
K = 16
NCHUNK = BPW // K
NTB = 782
TROWS = NTB * TW
TTR = TROWS // 8


def _sc_body(time_hbm, item_hbm, tf_hbm, if_hbm, out_hbm, ttab_hbm,
             tidx_v, iidx_v, tgidx_v, blk_v, trowbuf_v, trows_v, out_v,
             sem, sem_s, sem_t):
    sid = lax.axis_index("s")
    wid = sid * NC + lax.axis_index("c")
    base = wid * BPW
    lane = lax.iota(jnp.int32, L)

    pltpu.sync_copy(time_hbm.at[pl.ds(base, BPW)], tidx_v)
    pltpu.sync_copy(item_hbm.at[pl.ds(base, BPW)], iidx_v)

    nj = (NTB - 1) // NS + 1

    def fire_stage(j):
        bid = sid + j * NS
        slot = j & 1

        @pl.when(bid < NTB)
        def _():
            pltpu.async_copy(
                tf_hbm.at[pl.ds(0, F), pl.ds(bid * TW, TW)],
                blk_v.at[slot], sem_s)

    def stage_block(j, carry):
        fire_stage(j + 1)
        bid = sid + j * NS
        slot = j & 1

        @pl.when(bid < NTB)
        def _():
            pltpu.make_async_copy(
                tf_hbm.at[pl.ds(0, F), pl.ds(0, TW)], blk_v.at[slot], sem_s,
            ).wait()
            sv = jnp.full((L,), slot, jnp.int32)
            for lph in range(2):
                def lanebody(li, carry2):
                    l = lph * 64 + li
                    lv = jnp.full((L,), l, jnp.int32)
                    v1 = plsc.load_gather(blk_v, [sv, lane, lv])
                    v2 = plsc.load_gather(blk_v, [sv, lane + L, lv])
                    packed = plsc.pack(
                        v1, v2, format=plsc.PackFormat.INTERLEAVED)
                    trowbuf_v[li >> 3, pl.ds((li & 7) * L, L)] = (
                        plsc.bitcast(packed, jnp.int32))
                    return carry2

                lax.fori_loop(0, 64, lanebody, 0)
                pltpu.sync_copy(
                    trowbuf_v,
                    ttab_hbm.at[pl.ds(bid * (TW // 8) + lph * 8, 8)])
        return carry

    fire_stage(0)
    lax.fori_loop(0, nj, stage_block, 0)

    plsc.subcore_barrier()

    def tg(g, carry):
        s = pl.ds(g * L, L)
        tgidx_v[s] = tidx_v[s] >> 3
        return carry

    lax.fori_loop(0, BPW // L, tg, 0)

    def chunk(c, half, carry):
        idx_vec = iidx_v[pl.ds(c * K, L)]
        cols = (idx_vec >> 7) << 7
        lanes = idx_vec & (TW - 1)
        rows = (c - half * (NCHUNK // 2)) * K + lane
        tsub = (tidx_v[pl.ds(c * K, L)] & 7) << 4
        acc = jnp.zeros((L,), jnp.float32)
        for r in range(K):
            col = pl.multiple_of(cols[r], TW)
            pltpu.async_copy(
                if_hbm.at[pl.ds(0, F), pl.ds(col, TW)], blk_v.at[r], sem)

        def drain(r, carry2):
            pltpu.make_async_copy(
                if_hbm.at[pl.ds(0, F), pl.ds(0, TW)], blk_v.at[r], sem
            ).wait()
            return carry2

        lax.fori_loop(0, K, drain, 0, unroll=2)

        for f in range(F):
            fv = jnp.full((L,), f, jnp.int32)
            iv = plsc.load_gather(blk_v, [lane, fv, lanes])
            tp = plsc.load_gather(trows_v, [rows, tsub + (f % L)])
            t_lo, t_hi = plsc.unpack(
                plsc.bitcast(tp, jnp.bfloat16),
                format=plsc.PackFormat.INTERLEAVED)
            acc = acc + (t_lo if f < L else t_hi) * iv
        out_v[pl.ds(c * K, L)] = acc
        return carry

    for half in range(2):
        pltpu.async_copy(
            ttab_hbm.at[tgidx_v.at[pl.ds(half * (BPW // 2), BPW // 2)]],
            trows_v, sem_t).wait()
        lax.fori_loop(half * (NCHUNK // 2), (half + 1) * (NCHUNK // 2),
                      lambda c, carry, h=half: chunk(c, h, carry), 0)

    pltpu.sync_copy(out_v, out_hbm.at[pl.ds(base, BPW)])


@jax.jit
def _run(time, item, tfT, ifT):
    kern = pl.kernel(
        _sc_body,
        out_type=(
            jax.ShapeDtypeStruct((B,), jnp.float32),
            jax.ShapeDtypeStruct((TTR, 8 * L), jnp.int32),
        ),
        mesh=plsc.VectorSubcoreMesh(core_axis_name="c", subcore_axis_name="s"),
        compiler_params=pltpu.CompilerParams(needs_layout_passes=False),
        scratch_types=[
            pltpu.VMEM((BPW,), jnp.int32),
            pltpu.VMEM((BPW,), jnp.int32),
            pltpu.VMEM((BPW,), jnp.int32),
            pltpu.VMEM((K, F, TW), jnp.float32),
            pltpu.VMEM((8, 8 * L), jnp.int32),
            pltpu.VMEM((BPW // 2, 8 * L), jnp.int32),
            pltpu.VMEM((BPW,), jnp.float32),
            pltpu.SemaphoreType.DMA,
            pltpu.SemaphoreType.DMA,
            pltpu.SemaphoreType.DMA,
        ],
    )
    out, _ = kern(time, item, tfT, ifT)
    return out


def kernel(time, item, time_factor, item_factor, lag_factor):
    del lag_factor
    return _run(time.astype(jnp.int32), item.astype(jnp.int32),
                time_factor.T, item_factor.T)


# speedup vs baseline: 1.9275x; 1.5615x over previous
import functools

import jax
import jax.numpy as jnp
from jax import lax
from jax.experimental import pallas as pl
from jax.experimental.pallas import tpu as pltpu
from jax.experimental.pallas import tpu_sc as plsc

B = 16384
F = 32
FT = 4
L = 16
TW = 128
NC = 2
NS = 16
NW = NC * NS
BPW = B // NW
K = 16
NCHUNK = BPW // K
NTB = 782
TROWS = NTB * TW
TTR = TROWS // 8


def _sc_body(time_hbm, item_hbm, tf_hbm, if_hbm, out_hbm, ttab_hbm,
             tidx_v, iidx_v, tgidx_v, blk_v, trowbuf_v, trows_v, out_v,
             sem, sem_s, sem_t):
    sid = lax.axis_index("s")
    wid = sid * NC + lax.axis_index("c")
    base = wid * BPW
    lane = lax.iota(jnp.int32, L)

    pltpu.sync_copy(time_hbm.at[pl.ds(base, BPW)], tidx_v)
    pltpu.sync_copy(item_hbm.at[pl.ds(base, BPW)], iidx_v)

    nj = (NTB - 1) // NS + 1

    def fire_stage(j):
        bid = sid + j * NS
        slot = j & 1

        @pl.when(bid < NTB)
        def _():
            pltpu.async_copy(
                tf_hbm.at[pl.ds(0, F), pl.ds(bid * TW, TW)],
                blk_v.at[slot], sem_s)

    def stage_block(j, carry):
        fire_stage(j + 1)
        bid = sid + j * NS
        slot = j & 1

        @pl.when(bid < NTB)
        def _():
            pltpu.make_async_copy(
                tf_hbm.at[pl.ds(0, F), pl.ds(0, TW)], blk_v.at[slot], sem_s,
            ).wait()
            rhalf = lane >> 3
            colb = (lane & 7) * L
            for lph in range(2):
                for lgrp in range(4):
                    lbase = lph * 64 + lgrp * L
                    rowsv = lgrp * 2 + rhalf
                    for w in range(L):
                        a = blk_v[slot, w, pl.ds(lbase, L)]
                        b = blk_v[slot, w + L, pl.ds(lbase, L)]
                        words = plsc.bitcast(
                            plsc.pack(a, b,
                                      format=plsc.PackFormat.INTERLEAVED),
                            jnp.int32)
                        plsc.store_scatter(
                            trowbuf_v, [rowsv, colb + w], words)
                pltpu.sync_copy(
                    trowbuf_v,
                    ttab_hbm.at[pl.ds(bid * (TW // 8) + lph * 8, 8)])
        return carry

    fire_stage(0)
    lax.fori_loop(0, nj, stage_block, 0)

    plsc.subcore_barrier()

    def tg(g, carry):
        s = pl.ds(g * L, L)
        tgidx_v[s] = tidx_v[s] >> 3
        return carry

    lax.fori_loop(0, BPW // L, tg, 0)

    def chunk(c, half, carry):
        idx_vec = iidx_v[pl.ds(c * K, L)]
        cols = (idx_vec >> 7) << 7
        lanes = idx_vec & (TW - 1)
        rows = (c - half * (NCHUNK // 2)) * K + lane
        tsub = (tidx_v[pl.ds(c * K, L)] & 7) << 4
        acc = jnp.zeros((L,), jnp.float32)
        for r in range(K):
            col = pl.multiple_of(cols[r], TW)
            pltpu.async_copy(
                if_hbm.at[pl.ds(0, F), pl.ds(col, TW)], blk_v.at[r], sem)

        def drain(r, carry2):
            pltpu.make_async_copy(
                if_hbm.at[pl.ds(0, F), pl.ds(0, TW)], blk_v.at[r], sem
            ).wait()
            return carry2

        lax.fori_loop(0, K, drain, 0, unroll=2)

        for f in range(F):
            fv = jnp.full((L,), f, jnp.int32)
            iv = plsc.load_gather(blk_v, [lane, fv, lanes])
            tp = plsc.load_gather(trows_v, [rows, tsub + (f % L)])
            t_lo, t_hi = plsc.unpack(
                plsc.bitcast(tp, jnp.bfloat16),
                format=plsc.PackFormat.INTERLEAVED)
            acc = acc + (t_lo if f < L else t_hi) * iv
        out_v[pl.ds(c * K, L)] = acc
        return carry

    for half in range(2):
        pltpu.async_copy(
            ttab_hbm.at[tgidx_v.at[pl.ds(half * (BPW // 2), BPW // 2)]],
            trows_v, sem_t).wait()
        lax.fori_loop(half * (NCHUNK // 2), (half + 1) * (NCHUNK // 2),
                      lambda c, carry, h=half: chunk(c, h, carry), 0)

    pltpu.sync_copy(out_v, out_hbm.at[pl.ds(base, BPW)])


@jax.jit
def _run(time, item, tfT, ifT):
    kern = pl.kernel(
        _sc_body,
        out_type=(
            jax.ShapeDtypeStruct((B,), jnp.float32),
            jax.ShapeDtypeStruct((TTR, 8 * L), jnp.int32),
        ),
        mesh=plsc.VectorSubcoreMesh(core_axis_name="c", subcore_axis_name="s"),
        compiler_params=pltpu.CompilerParams(needs_layout_passes=False),
        scratch_types=[
            pltpu.VMEM((BPW,), jnp.int32),
            pltpu.VMEM((BPW,), jnp.int32),
            pltpu.VMEM((BPW,), jnp.int32),
            pltpu.VMEM((K, F, TW), jnp.float32),
            pltpu.VMEM((8, 8 * L), jnp.int32),
            pltpu.VMEM((BPW // 2, 8 * L), jnp.int32),
            pltpu.VMEM((BPW,), jnp.float32),
            pltpu.SemaphoreType.DMA,
            pltpu.SemaphoreType.DMA,
            pltpu.SemaphoreType.DMA,
        ],
    )
    out, _ = kern(time, item, tfT, ifT)
    return out


def kernel(time, item, time_factor, item_factor, lag_factor):
    del lag_factor
    return _run(time.astype(jnp.int32), item.astype(jnp.int32),
                time_factor.T, item_factor.T)
